# Initial kernel scaffold; baseline (speedup 1.0000x reference)
import functools

import jax
import jax.numpy as jnp
from jax import lax
from jax.experimental import pallas as pl
from jax.experimental.pallas import tpu as pltpu
from jax.experimental.pallas import tpu_sc as plsc

N = 10000
D = 128
E = 320000
M = 3

NC = 2
NS = 16
NW = NC * NS
NPAD = 10240
RPT = NPAD // NS
K = 128
WTOT = E // K
WBASE = WTOT // NW
WREM = WTOT - WBASE * NW

BN = 1024
NB = NPAD // BN


def _sc_body(f0, f1, f2, s0, d0, s1, d1, s2, d2, zrows, zdeg,
             acc_out, deg_out, src_v, dst_v, rows_v, degl_v, acc_sh, sem):
    c = lax.axis_index("c")
    s = lax.axis_index("s")
    wid = c * NS + s
    base_row = s * RPT
    feats = (f0, f1, f2)
    srcs = (s0, s1, s2)
    dsts = (d0, d1, d2)
    nw_me = jnp.where(wid < WREM, WBASE + 1, WBASE)
    w0 = wid * WBASE + jnp.minimum(wid, WREM)
    ones16 = jnp.ones((16,), jnp.float32)

    for m in range(M):
        pltpu.sync_copy(zrows, acc_sh.at[pl.ds(base_row, RPT)])
        pltpu.sync_copy(zdeg, degl_v)
        plsc.subcore_barrier()

        def window(j, carry):
            off = (w0 + j) * K
            pltpu.sync_copy(srcs[m].at[pl.ds(off, K)], src_v)
            pltpu.sync_copy(dsts[m].at[pl.ds(off, K)], dst_v)
            pltpu.async_copy(feats[m].at[src_v], rows_v, sem).wait()
            pltpu.sync_copy(rows_v, acc_sh.at[dst_v], add=True)
            for j2 in range(K // 16):
                idx = dst_v[pl.ds(j2 * 16, 16)]
                plsc.addupdate_scatter(degl_v, [idx], ones16)
            return carry

        lax.fori_loop(0, nw_me, window, 0)
        plsc.subcore_barrier()

        pltpu.sync_copy(acc_sh.at[pl.ds(base_row, RPT)],
                        acc_out.at[m, c, pl.ds(base_row, RPT)])
        pltpu.sync_copy(degl_v, deg_out.at[m, wid])
        plsc.subcore_barrier()


def _sc_aggregate(feats, srcs, dsts):
    mesh = plsc.VectorSubcoreMesh(core_axis_name="c", subcore_axis_name="s",
                                  num_cores=NC, num_subcores=NS)
    zrows = jnp.zeros((RPT, D), jnp.float32)
    zdeg = jnp.zeros((NPAD,), jnp.float32)
    fn = pl.kernel(
        _sc_body,
        out_type=(jax.ShapeDtypeStruct((M, NC, NPAD, D), jnp.float32),
                  jax.ShapeDtypeStruct((M, NW, NPAD), jnp.float32)),
        mesh=mesh,
        scratch_types=[
            pltpu.VMEM((K,), jnp.int32),
            pltpu.VMEM((K,), jnp.int32),
            pltpu.VMEM((K, D), jnp.float32),
            pltpu.VMEM((NPAD,), jnp.float32),
            pltpu.VMEM_SHARED((NPAD, D), jnp.float32),
            pltpu.SemaphoreType.DMA,
        ],
    )
    return fn(feats[0], feats[1], feats[2],
              srcs[0], dsts[0], srcs[1], dsts[1], srcs[2], dsts[2],
              zrows, zdeg)


def _dense_body(acc_ref, deg_ref, w_ref, b_ref, a_ref, wa_ref, ba_ref,
                h_ref, s_ref):
    bi = pl.program_id(1)
    acc = acc_ref[0, 0] + acc_ref[0, 1]
    degb = deg_ref[0]
    degc = lax.dot_general(degb, jnp.ones((NW, 1), jnp.float32),
                           dimension_numbers=(((0,), (0,)), ((), ())),
                           preferred_element_type=jnp.float32)
    degc = jnp.maximum(degc, 1.0)
    y = jnp.dot(acc, w_ref[0], preferred_element_type=jnp.float32)
    h = y / degc + b_ref[0]
    a = a_ref[0, 0]
    h = jnp.maximum(h, 0.0) + a * jnp.minimum(h, 0.0)
    h_ref[0] = h
    t = jnp.tanh(jnp.dot(h, wa_ref[...], preferred_element_type=jnp.float32)
                 + ba_ref[...])
    rows = lax.broadcasted_iota(jnp.int32, (BN, 1), 0) + bi * BN
    t = jnp.where(rows < N, t, 0.0)
    part = jnp.sum(t, axis=0, keepdims=True)

    @pl.when(bi == 0)
    def _():
        s_ref[0] = part

    @pl.when(bi != 0)
    def _():
        s_ref[0] = s_ref[0] + part


def _dense_stage(acc, deg, wstk, bstk, astk, wa, ba):
    return pl.pallas_call(
        _dense_body,
        grid=(M, NB),
        in_specs=[
            pl.BlockSpec((1, NC, BN, D), lambda m, b: (m, 0, b, 0)),
            pl.BlockSpec((1, NW, BN), lambda m, b: (m, 0, b)),
            pl.BlockSpec((1, D, D), lambda m, b: (m, 0, 0)),
            pl.BlockSpec((1, 1, D), lambda m, b: (m, 0, 0)),
            pl.BlockSpec((1, 1, 1), lambda m, b: (m, 0, 0)),
            pl.BlockSpec((D, D), lambda m, b: (0, 0)),
            pl.BlockSpec((1, D), lambda m, b: (0, 0)),
        ],
        out_specs=[
            pl.BlockSpec((1, BN, D), lambda m, b: (m, b, 0)),
            pl.BlockSpec((1, 1, D), lambda m, b: (m, 0, 0)),
        ],
        out_shape=[
            jax.ShapeDtypeStruct((M, NPAD, D), jnp.float32),
            jax.ShapeDtypeStruct((M, 1, D), jnp.float32),
        ],
    )(acc, deg, wstk, bstk, astk, wa, ba)


def _mix_body(s_ref, av_ref, h_ref, z_ref):
    sm = s_ref[...].reshape(M, D) * jnp.float32(1.0 / N)
    w = jnp.sum(sm * av_ref[...], axis=1, keepdims=True)
    w = w - jnp.max(w)
    e = jnp.exp(w)
    beta = e / jnp.sum(e)
    z = (h_ref[0] * beta[0:1, 0:1]
         + h_ref[1] * beta[1:2, 0:1]
         + h_ref[2] * beta[2:3, 0:1])
    z_ref[...] = z


def _mix_stage(sstat, av, h):
    return pl.pallas_call(
        _mix_body,
        grid=(NB,),
        in_specs=[
            pl.BlockSpec((M, 1, D), lambda b: (0, 0, 0)),
            pl.BlockSpec((1, D), lambda b: (0, 0)),
            pl.BlockSpec((M, BN, D), lambda b: (0, b, 0)),
        ],
        out_specs=pl.BlockSpec((BN, D), lambda b: (b, 0)),
        out_shape=jax.ShapeDtypeStruct((NPAD, D), jnp.float32),
    )(sstat, av, h)


def kernel(feat0, feat1, feat2, edge_index0, edge_index1, edge_index2,
           W0, b0, prelu_a0, W1, b1, prelu_a1, W2, b2, prelu_a2,
           attn_fc_W, attn_fc_b, attn_vec):
    feats = (feat0, feat1, feat2)
    srcs = (edge_index0[0], edge_index1[0], edge_index2[0])
    dsts = (edge_index0[1], edge_index1[1], edge_index2[1])

    acc, deg = _sc_aggregate(feats, srcs, dsts)

    wstk = jnp.stack([W0, W1, W2])
    bstk = jnp.stack([b0, b1, b2]).reshape(M, 1, D)
    astk = jnp.stack([prelu_a0, prelu_a1, prelu_a2]).reshape(M, 1, 1)
    ba = attn_fc_b.reshape(1, D)

    h, sstat = _dense_stage(acc, deg, wstk, bstk, astk, attn_fc_W, ba)
    z = _mix_stage(sstat, attn_vec, h)
    return z[:N]


# speedup vs baseline: 5.3551x; 5.3551x over previous
import functools

import jax
import jax.numpy as jnp
from jax import lax
from jax.experimental import pallas as pl
from jax.experimental.pallas import tpu as pltpu
from jax.experimental.pallas import tpu_sc as plsc

N = 10000
D = 128
E = 320000
M = 3

NC = 2
NS = 16
NW = NC * NS
NPAD = 10240
RPT = NPAD // NS
K = 128
WTOT = E // K
WBASE = WTOT // NW
WREM = WTOT - WBASE * NW

BN = 1024
NB = NPAD // BN


def _sc_body(f0, f1, f2, s0, d0, s1, d1, s2, d2, zrows, zdeg, ones_h,
             acc_out, deg_out, src_v, dst_v, rows_v, ones_v, acc_sh, deg_sh,
             sem):
    c = lax.axis_index("c")
    s = lax.axis_index("s")
    wid = c * NS + s
    base_row = s * RPT
    feats = (f0, f1, f2)
    srcs = (s0, s1, s2)
    dsts = (d0, d1, d2)
    nw_me = jnp.where(wid < WREM, WBASE + 1, WBASE)
    w0 = wid * WBASE + jnp.minimum(wid, WREM)
    pltpu.sync_copy(ones_h, ones_v)

    for m in range(M):
        pltpu.sync_copy(zrows, acc_sh.at[pl.ds(base_row, RPT)])
        pltpu.sync_copy(zdeg.at[pl.ds(base_row, RPT)],
                        deg_sh.at[pl.ds(base_row, RPT)])
        plsc.subcore_barrier()

        def window(j, carry):
            off = (w0 + j) * K
            pltpu.sync_copy(srcs[m].at[pl.ds(off, K)], src_v)
            pltpu.sync_copy(dsts[m].at[pl.ds(off, K)], dst_v)
            pltpu.async_copy(feats[m].at[src_v], rows_v, sem).wait()
            pltpu.sync_copy(rows_v, acc_sh.at[dst_v], add=True)
            pltpu.sync_copy(ones_v, deg_sh.at[dst_v], add=True)
            return carry

        lax.fori_loop(0, nw_me, window, 0)
        plsc.subcore_barrier()

        pltpu.sync_copy(acc_sh.at[pl.ds(base_row, RPT)],
                        acc_out.at[m, c, pl.ds(base_row, RPT)])
        pltpu.sync_copy(deg_sh.at[pl.ds(base_row, RPT)],
                        deg_out.at[m, c, pl.ds(base_row, RPT)])
        plsc.subcore_barrier()


def _sc_aggregate(feats, srcs, dsts):
    mesh = plsc.VectorSubcoreMesh(core_axis_name="c", subcore_axis_name="s",
                                  num_cores=NC, num_subcores=NS)
    zrows = jnp.zeros((RPT, D), jnp.float32)
    zdeg = jnp.zeros((NPAD,), jnp.float32)
    ones_h = jnp.ones((K,), jnp.float32)
    fn = pl.kernel(
        _sc_body,
        out_type=(jax.ShapeDtypeStruct((M, NC, NPAD, D), jnp.float32),
                  jax.ShapeDtypeStruct((M, NC, NPAD), jnp.float32)),
        mesh=mesh,
        scratch_types=[
            pltpu.VMEM((K,), jnp.int32),
            pltpu.VMEM((K,), jnp.int32),
            pltpu.VMEM((K, D), jnp.float32),
            pltpu.VMEM((K,), jnp.float32),
            pltpu.VMEM_SHARED((NPAD, D), jnp.float32),
            pltpu.VMEM_SHARED((NPAD,), jnp.float32),
            pltpu.SemaphoreType.DMA,
        ],
    )
    return fn(feats[0], feats[1], feats[2],
              srcs[0], dsts[0], srcs[1], dsts[1], srcs[2], dsts[2],
              zrows, zdeg, ones_h)


def _dense_body(acc_ref, deg_ref, w_ref, b_ref, a_ref, wa_ref, ba_ref,
                h_ref, s_ref):
    bi = pl.program_id(1)
    acc = acc_ref[0, 0] + acc_ref[0, 1]
    degb = deg_ref[0]
    degc = lax.dot_general(degb, jnp.ones((NC, 1), jnp.float32),
                           dimension_numbers=(((0,), (0,)), ((), ())),
                           preferred_element_type=jnp.float32)
    degc = jnp.maximum(degc, 1.0)
    y = jnp.dot(acc, w_ref[0], preferred_element_type=jnp.float32)
    h = y / degc + b_ref[0]
    a = a_ref[0, 0]
    h = jnp.maximum(h, 0.0) + a * jnp.minimum(h, 0.0)
    h_ref[0] = h
    t = jnp.tanh(jnp.dot(h, wa_ref[...], preferred_element_type=jnp.float32)
                 + ba_ref[...])
    rows = lax.broadcasted_iota(jnp.int32, (BN, 1), 0) + bi * BN
    t = jnp.where(rows < N, t, 0.0)
    part = jnp.sum(t, axis=0, keepdims=True)

    @pl.when(bi == 0)
    def _():
        s_ref[0] = part

    @pl.when(bi != 0)
    def _():
        s_ref[0] = s_ref[0] + part


def _dense_stage(acc, deg, wstk, bstk, astk, wa, ba):
    return pl.pallas_call(
        _dense_body,
        grid=(M, NB),
        in_specs=[
            pl.BlockSpec((1, NC, BN, D), lambda m, b: (m, 0, b, 0)),
            pl.BlockSpec((1, NC, BN), lambda m, b: (m, 0, b)),
            pl.BlockSpec((1, D, D), lambda m, b: (m, 0, 0)),
            pl.BlockSpec((1, 1, D), lambda m, b: (m, 0, 0)),
            pl.BlockSpec((1, 1, 1), lambda m, b: (m, 0, 0)),
            pl.BlockSpec((D, D), lambda m, b: (0, 0)),
            pl.BlockSpec((1, D), lambda m, b: (0, 0)),
        ],
        out_specs=[
            pl.BlockSpec((1, BN, D), lambda m, b: (m, b, 0)),
            pl.BlockSpec((1, 1, D), lambda m, b: (m, 0, 0)),
        ],
        out_shape=[
            jax.ShapeDtypeStruct((M, NPAD, D), jnp.float32),
            jax.ShapeDtypeStruct((M, 1, D), jnp.float32),
        ],
    )(acc, deg, wstk, bstk, astk, wa, ba)


def _mix_body(s_ref, av_ref, h_ref, z_ref):
    sm = s_ref[...].reshape(M, D) * jnp.float32(1.0 / N)
    w = jnp.sum(sm * av_ref[...], axis=1, keepdims=True)
    w = w - jnp.max(w)
    e = jnp.exp(w)
    beta = e / jnp.sum(e)
    z = (h_ref[0] * beta[0:1, 0:1]
         + h_ref[1] * beta[1:2, 0:1]
         + h_ref[2] * beta[2:3, 0:1])
    z_ref[...] = z


def _mix_stage(sstat, av, h):
    return pl.pallas_call(
        _mix_body,
        grid=(NB,),
        in_specs=[
            pl.BlockSpec((M, 1, D), lambda b: (0, 0, 0)),
            pl.BlockSpec((1, D), lambda b: (0, 0)),
            pl.BlockSpec((M, BN, D), lambda b: (0, b, 0)),
        ],
        out_specs=pl.BlockSpec((BN, D), lambda b: (b, 0)),
        out_shape=jax.ShapeDtypeStruct((NPAD, D), jnp.float32),
    )(sstat, av, h)


def kernel(feat0, feat1, feat2, edge_index0, edge_index1, edge_index2,
           W0, b0, prelu_a0, W1, b1, prelu_a1, W2, b2, prelu_a2,
           attn_fc_W, attn_fc_b, attn_vec):
    feats = (feat0, feat1, feat2)
    srcs = (edge_index0[0], edge_index1[0], edge_index2[0])
    dsts = (edge_index0[1], edge_index1[1], edge_index2[1])

    acc, deg = _sc_aggregate(feats, srcs, dsts)

    wstk = jnp.stack([W0, W1, W2])
    bstk = jnp.stack([b0, b1, b2]).reshape(M, 1, D)
    astk = jnp.stack([prelu_a0, prelu_a1, prelu_a2]).reshape(M, 1, 1)
    ba = attn_fc_b.reshape(1, D)

    h, sstat = _dense_stage(acc, deg, wstk, bstk, astk, attn_fc_W, ba)
    z = _mix_stage(sstat, attn_vec, h)
    return z[:N]


# speedup vs baseline: 8.3360x; 1.5566x over previous
import functools

import jax
import jax.numpy as jnp
from jax import lax
from jax.experimental import pallas as pl
from jax.experimental.pallas import tpu as pltpu
from jax.experimental.pallas import tpu_sc as plsc

N = 10000
D = 128
E = 320000
M = 3

NC = 2
NS = 16
NW = NC * NS
NPAD = 10240
RPT = NPAD // NS
K = 80
WTOT = E // K
WPW = WTOT // NW

BN = 1024
NB = NPAD // BN


def _sc_body(f0, f1, f2, s0, d0, s1, d1, s2, d2, zrows, zdeg, ones_h,
             acc_out, deg_out, sbuf0, dbuf0, sbuf1, dbuf1, rows0, rows1,
             ones_v, acc_sh, deg_sh, isem0, isem1, gsem0, gsem1, dsem0, dsem1):
    c = lax.axis_index("c")
    s = lax.axis_index("s")
    wid = c * NS + s
    base_row = s * RPT
    feats = (f0, f1, f2)
    srcs = (s0, s1, s2)
    dsts = (d0, d1, d2)
    w0 = wid * WPW
    pltpu.sync_copy(ones_h, ones_v)

    for m in range(M):
        feat, src, dst = feats[m], srcs[m], dsts[m]

        def fetch_idx(j, sb, db, isem):
            off = jnp.minimum(w0 + j, WTOT - 1) * K
            pltpu.async_copy(src.at[pl.ds(off, K)], sb, isem)
            pltpu.async_copy(dst.at[pl.ds(off, K)], db, isem)

        def wait_idx(sb, db, isem):
            pltpu.make_async_copy(src.at[pl.ds(0, K)], sb, isem).wait()
            pltpu.make_async_copy(dst.at[pl.ds(0, K)], db, isem).wait()

        def wait_rows(rb, gsem):
            pltpu.make_async_copy(feat.at[pl.ds(0, K)], rb, gsem).wait()

        def scat(rb, db, gsem, dsem):
            wait_rows(rb, gsem)
            pltpu.async_copy(ones_v, deg_sh.at[db], dsem, add=True)
            pltpu.sync_copy(rb, acc_sh.at[db], add=True)
            pltpu.make_async_copy(ones_v, deg_sh.at[pl.ds(0, K)], dsem).wait()

        pltpu.sync_copy(zrows, acc_sh.at[pl.ds(base_row, RPT)])
        pltpu.sync_copy(zdeg.at[pl.ds(base_row, RPT)],
                        deg_sh.at[pl.ds(base_row, RPT)])
        plsc.subcore_barrier()

        fetch_idx(0, sbuf0, dbuf0, isem0)
        fetch_idx(1, sbuf1, dbuf1, isem1)
        wait_idx(sbuf0, dbuf0, isem0)
        pltpu.async_copy(feat.at[sbuf0], rows0, gsem0)

        def pair(i, carry):
            a = 2 * i
            wait_idx(sbuf1, dbuf1, isem1)
            pltpu.async_copy(feat.at[sbuf1], rows1, gsem1)
            scat(rows0, dbuf0, gsem0, dsem0)
            fetch_idx(a + 2, sbuf0, dbuf0, isem0)
            scat(rows1, dbuf1, gsem1, dsem1)
            fetch_idx(a + 3, sbuf1, dbuf1, isem1)
            wait_idx(sbuf0, dbuf0, isem0)
            pltpu.async_copy(feat.at[sbuf0], rows0, gsem0)
            return carry

        lax.fori_loop(0, (WPW - 1) // 2, pair, 0)
        wait_idx(sbuf1, dbuf1, isem1)
        scat(rows0, dbuf0, gsem0, dsem0)
        plsc.subcore_barrier()

        pltpu.sync_copy(acc_sh.at[pl.ds(base_row, RPT)],
                        acc_out.at[m, c, pl.ds(base_row, RPT)])
        pltpu.sync_copy(deg_sh.at[pl.ds(base_row, RPT)],
                        deg_out.at[m, c, pl.ds(base_row, RPT)])
        plsc.subcore_barrier()


def _sc_aggregate(feats, srcs, dsts):
    mesh = plsc.VectorSubcoreMesh(core_axis_name="c", subcore_axis_name="s",
                                  num_cores=NC, num_subcores=NS)
    zrows = jnp.zeros((RPT, D), jnp.float32)
    zdeg = jnp.zeros((NPAD,), jnp.float32)
    ones_h = jnp.ones((K,), jnp.float32)
    fn = pl.kernel(
        _sc_body,
        out_type=(jax.ShapeDtypeStruct((M, NC, NPAD, D), jnp.float32),
                  jax.ShapeDtypeStruct((M, NC, NPAD), jnp.float32)),
        mesh=mesh,
        scratch_types=[
            pltpu.VMEM((K,), jnp.int32),
            pltpu.VMEM((K,), jnp.int32),
            pltpu.VMEM((K,), jnp.int32),
            pltpu.VMEM((K,), jnp.int32),
            pltpu.VMEM((K, D), jnp.float32),
            pltpu.VMEM((K, D), jnp.float32),
            pltpu.VMEM((K,), jnp.float32),
            pltpu.VMEM_SHARED((NPAD, D), jnp.float32),
            pltpu.VMEM_SHARED((NPAD,), jnp.float32),
            pltpu.SemaphoreType.DMA,
            pltpu.SemaphoreType.DMA,
            pltpu.SemaphoreType.DMA,
            pltpu.SemaphoreType.DMA,
            pltpu.SemaphoreType.DMA,
            pltpu.SemaphoreType.DMA,
        ],
    )
    return fn(feats[0], feats[1], feats[2],
              srcs[0], dsts[0], srcs[1], dsts[1], srcs[2], dsts[2],
              zrows, zdeg, ones_h)


def _dense_body(acc_ref, deg_ref, w_ref, b_ref, a_ref, wa_ref, ba_ref,
                h_ref, s_ref):
    bi = pl.program_id(1)
    acc = acc_ref[0, 0] + acc_ref[0, 1]
    degb = deg_ref[0]
    degc = lax.dot_general(degb, jnp.ones((NC, 1), jnp.float32),
                           dimension_numbers=(((0,), (0,)), ((), ())),
                           preferred_element_type=jnp.float32)
    degc = jnp.maximum(degc, 1.0)
    y = jnp.dot(acc, w_ref[0], preferred_element_type=jnp.float32)
    h = y / degc + b_ref[0]
    a = a_ref[0, 0]
    h = jnp.maximum(h, 0.0) + a * jnp.minimum(h, 0.0)
    h_ref[0] = h
    t = jnp.tanh(jnp.dot(h, wa_ref[...], preferred_element_type=jnp.float32)
                 + ba_ref[...])
    rows = lax.broadcasted_iota(jnp.int32, (BN, 1), 0) + bi * BN
    t = jnp.where(rows < N, t, 0.0)
    part = jnp.sum(t, axis=0, keepdims=True)

    @pl.when(bi == 0)
    def _():
        s_ref[0] = part

    @pl.when(bi != 0)
    def _():
        s_ref[0] = s_ref[0] + part


def _dense_stage(acc, deg, wstk, bstk, astk, wa, ba):
    return pl.pallas_call(
        _dense_body,
        grid=(M, NB),
        in_specs=[
            pl.BlockSpec((1, NC, BN, D), lambda m, b: (m, 0, b, 0)),
            pl.BlockSpec((1, NC, BN), lambda m, b: (m, 0, b)),
            pl.BlockSpec((1, D, D), lambda m, b: (m, 0, 0)),
            pl.BlockSpec((1, 1, D), lambda m, b: (m, 0, 0)),
            pl.BlockSpec((1, 1, 1), lambda m, b: (m, 0, 0)),
            pl.BlockSpec((D, D), lambda m, b: (0, 0)),
            pl.BlockSpec((1, D), lambda m, b: (0, 0)),
        ],
        out_specs=[
            pl.BlockSpec((1, BN, D), lambda m, b: (m, b, 0)),
            pl.BlockSpec((1, 1, D), lambda m, b: (m, 0, 0)),
        ],
        out_shape=[
            jax.ShapeDtypeStruct((M, NPAD, D), jnp.float32),
            jax.ShapeDtypeStruct((M, 1, D), jnp.float32),
        ],
    )(acc, deg, wstk, bstk, astk, wa, ba)


def _mix_body(s_ref, av_ref, h_ref, z_ref):
    sm = s_ref[...].reshape(M, D) * jnp.float32(1.0 / N)
    w = jnp.sum(sm * av_ref[...], axis=1, keepdims=True)
    w = w - jnp.max(w)
    e = jnp.exp(w)
    beta = e / jnp.sum(e)
    z = (h_ref[0] * beta[0:1, 0:1]
         + h_ref[1] * beta[1:2, 0:1]
         + h_ref[2] * beta[2:3, 0:1])
    z_ref[...] = z


def _mix_stage(sstat, av, h):
    return pl.pallas_call(
        _mix_body,
        grid=(NB,),
        in_specs=[
            pl.BlockSpec((M, 1, D), lambda b: (0, 0, 0)),
            pl.BlockSpec((1, D), lambda b: (0, 0)),
            pl.BlockSpec((M, BN, D), lambda b: (0, b, 0)),
        ],
        out_specs=pl.BlockSpec((BN, D), lambda b: (b, 0)),
        out_shape=jax.ShapeDtypeStruct((NPAD, D), jnp.float32),
    )(sstat, av, h)


def kernel(feat0, feat1, feat2, edge_index0, edge_index1, edge_index2,
           W0, b0, prelu_a0, W1, b1, prelu_a1, W2, b2, prelu_a2,
           attn_fc_W, attn_fc_b, attn_vec):
    feats = (feat0, feat1, feat2)
    srcs = tuple(e[0] for e in (edge_index0, edge_index1, edge_index2))
    dsts = tuple(e[1] for e in (edge_index0, edge_index1, edge_index2))

    acc, deg = _sc_aggregate(feats, srcs, dsts)

    wstk = jnp.stack([W0, W1, W2])
    bstk = jnp.stack([b0, b1, b2]).reshape(M, 1, D)
    astk = jnp.stack([prelu_a0, prelu_a1, prelu_a2]).reshape(M, 1, 1)
    ba = attn_fc_b.reshape(1, D)

    h, sstat = _dense_stage(acc, deg, wstk, bstk, astk, attn_fc_W, ba)
    z = _mix_stage(sstat, attn_vec, h)
    return z[:N]
